# Initial kernel scaffold; baseline (speedup 1.0000x reference)
import jax
import jax.numpy as jnp
from jax.experimental import pallas as pl


def kernel(x, edge_index, batch, y, W0, b0, W1, b1, W2, b2, Ws0, bs0, Ws1, bs1, Ws2, bs2, Wf, bf, Wm0, bm0, Wm1, bm1, Wm2, bm2, Wm3, bm3):
    raise NotImplementedError("write your pallas kernel here")


# speedup vs baseline: 1.0000x; 1.0000x over previous
import jax
import jax.numpy as jnp
from jax.experimental import pallas as pl

N = 10000
E = 320000
G = 64
RATIO = 0.5


def _gcn(x, src, dst, W, b, alive):
    a = alive.astype(x.dtype)
    em = a[src] * a[dst]
    deg = jax.ops.segment_sum(em, dst, num_segments=N) + a
    dinv = jnp.where(deg > 0, 1.0 / jnp.sqrt(jnp.maximum(deg, 1e-12)), 0.0)
    h = x @ W
    norm = dinv[src] * dinv[dst] * em
    out = jax.ops.segment_sum(h[src] * norm[:, None], dst, num_segments=N)
    out = out + h * (dinv * dinv * a)[:, None]
    return out + b


def _sag_pool(x, src, dst, batch, alive, Ws, bs):
    score = _gcn(x, src, dst, Ws, bs, alive)[:, 0]
    s_eff = jnp.where(alive, score, -1e30)
    cnt_alive = jax.ops.segment_sum(alive.astype(jnp.float32), batch, num_segments=G)
    k_g = jnp.ceil(RATIO * cnt_alive)
    perm = jnp.lexsort((-s_eff, batch))
    inv = jnp.zeros((N,), jnp.int32).at[perm].set(jnp.arange(N, dtype=jnp.int32))
    cnt_all = jax.ops.segment_sum(jnp.ones((N,), jnp.int32), batch, num_segments=G)
    starts = jnp.cumsum(cnt_all) - cnt_all
    rank = inv - starts[batch]
    keep = alive & (rank.astype(jnp.float32) < k_g[batch])
    x_new = jnp.where(keep[:, None], x * jnp.tanh(score)[:, None], 0.0)
    return x_new, keep


def _readout(x, batch, alive):
    add = jax.ops.segment_sum(x, batch, num_segments=G)
    mx = jax.ops.segment_max(jnp.where(alive[:, None], x, -1e30), batch, num_segments=G)
    cnt = jax.ops.segment_sum(alive.astype(jnp.float32), batch, num_segments=G)
    mx = jnp.where(cnt[:, None] > 0, mx, 0.0)
    return jnp.concatenate([mx, add], axis=1)


def kernel(x, edge_index, batch, y, W0, b0, W1, b1, W2, b2, Ws0, bs0, Ws1, bs1, Ws2, bs2, Wf, bf, Wm0, bm0, Wm1, bm1, Wm2, bm2, Wm3, bm3):
    src = edge_index[0]
    dst = edge_index[1]
    alive = jnp.ones((N,), dtype=bool)
    convs = [(W0, b0), (W1, b1), (W2, b2)]
    pools = [(Ws0, bs0), (Ws1, bs1), (Ws2, bs2)]
    hs = []
    for (Wc, bc), (Ws, bs) in zip(convs, pools):
        x = jax.nn.relu(_gcn(x, src, dst, Wc, bc, alive))
        x, alive = _sag_pool(x, src, dst, batch, alive, Ws, bs)
        hs.append(_readout(x, batch, alive))
    z = hs[0] + hs[1] + hs[2]
    z = z @ Wf + bf
    z = jax.nn.relu(z @ Wm0 + bm0)
    z = jax.nn.relu(z @ Wm1 + bm1)
    z = jax.nn.relu(z @ Wm2 + bm2)
    out = z @ Wm3 + bm3
    target = y.reshape((G, 1))
    total_loss = jnp.sqrt(jnp.mean((out - target) ** 2))
    return (out, total_loss)


# speedup vs baseline: 15.5064x; 15.5062x over previous
import functools

import jax
import jax.numpy as jnp
from jax import lax
from jax.experimental import pallas as pl
from jax.experimental.pallas import tpu as pltpu
from jax.experimental.pallas import tpu_sc as plsc

N = 10000
NP = 10240
E = 320000
EPW = 10112
EP = EPW * 32
NCH = EPW // 128
G = 64
H = 128
RATIO = 0.5
JC = 256
NJC = NP // JC


def _sc_agg_rows_body(p_hbm, srcr, dstr, out_hbm, src_v, dst_v, rows_v, acc_sh, sem):
    cid = lax.axis_index("c")
    sid = lax.axis_index("s")
    wid = cid * 16 + sid
    z16 = jnp.zeros((16,), jnp.float32)

    def zrow(i, _):
        for j in range(8):
            rows_v[i, j * 16:(j + 1) * 16] = z16
        return 0

    lax.fori_loop(0, 128, zrow, 0)
    for t in range(5):
        pltpu.sync_copy(rows_v, acc_sh.at[pl.ds(sid * 640 + t * 128, 128)])
    plsc.subcore_barrier()

    pltpu.sync_copy(srcr.at[wid], src_v)
    pltpu.sync_copy(dstr.at[wid], dst_v)

    def body(c, _):
        pltpu.async_copy(p_hbm.at[src_v.at[c]], rows_v, sem).wait()
        pltpu.sync_copy(rows_v, acc_sh.at[dst_v.at[c]], add=True)
        return 0

    lax.fori_loop(0, NCH, body, 0)
    plsc.subcore_barrier()
    for t in range(5):
        pltpu.sync_copy(acc_sh.at[pl.ds(sid * 640 + t * 128, 128)],
                        out_hbm.at[cid, pl.ds(sid * 640 + t * 128, 128)])


@functools.lru_cache(maxsize=None)
def _sc_agg_rows_call():
    mesh = plsc.VectorSubcoreMesh(core_axis_name="c", subcore_axis_name="s")
    return pl.kernel(
        _sc_agg_rows_body,
        out_type=jax.ShapeDtypeStruct((2, NP, H), jnp.float32),
        mesh=mesh,
        compiler_params=pltpu.CompilerParams(needs_layout_passes=False),
        scratch_types=[
            pltpu.VMEM((NCH, 128), jnp.int32),
            pltpu.VMEM((NCH, 128), jnp.int32),
            pltpu.VMEM((128, H), jnp.float32),
            pltpu.VMEM_SHARED((NP, H), jnp.float32),
            pltpu.SemaphoreType.DMA,
        ],
    )


def _sc_agg_rows(p, srcr, dstr):
    return _sc_agg_rows_call()(p, srcr, dstr)


def _sc_seg_scalar_body(vals_hbm, srcr, dstr, out_hbm, vals_v, src_v, dst_v,
                        part_v, tmp_v, red_v, stage_sh):
    cid = lax.axis_index("c")
    sid = lax.axis_index("s")
    wid = cid * 16 + sid
    z16 = jnp.zeros((16,), jnp.float32)

    def zpart(i, _):
        for j in range(8):
            part_v[i, j * 16:(j + 1) * 16] = z16
        return 0

    lax.fori_loop(0, NP // 128, zpart, 0)
    pltpu.sync_copy(vals_hbm, vals_v)
    pltpu.sync_copy(srcr.at[wid], src_v)
    pltpu.sync_copy(dstr.at[wid], dst_v)

    def body(c, _):
        for j in range(8):
            s16 = src_v[c, j * 16:(j + 1) * 16]
            d16 = dst_v[c, j * 16:(j + 1) * 16]
            v = plsc.load_gather(
                vals_v, [jnp.right_shift(s16, 7), jnp.bitwise_and(s16, 127)])
            plsc.addupdate_scatter(
                part_v, [jnp.right_shift(d16, 7), jnp.bitwise_and(d16, 127)], v)
        return 0

    lax.fori_loop(0, NCH, body, 0)
    pltpu.sync_copy(part_v, stage_sh.at[sid])
    plsc.subcore_barrier()

    @pl.when(sid < 10)
    def _():
        for i in range(8):
            for j in range(8):
                red_v[i, j * 16:(j + 1) * 16] = z16

        def comb(t, _):
            pltpu.sync_copy(stage_sh.at[t, pl.ds(sid * 8, 8)], tmp_v)
            for i in range(8):
                for j in range(8):
                    sl = (i, slice(j * 16, (j + 1) * 16))
                    red_v[sl] = red_v[sl] + tmp_v[sl]
            return 0

        lax.fori_loop(0, 16, comb, 0)
        pltpu.sync_copy(red_v, out_hbm.at[cid, pl.ds(sid * 8, 8)])


@functools.lru_cache(maxsize=None)
def _sc_seg_scalar_call():
    mesh = plsc.VectorSubcoreMesh(core_axis_name="c", subcore_axis_name="s")
    return pl.kernel(
        _sc_seg_scalar_body,
        out_type=jax.ShapeDtypeStruct((2, NP // 128, 128), jnp.float32),
        mesh=mesh,
        compiler_params=pltpu.CompilerParams(needs_layout_passes=False),
        scratch_types=[
            pltpu.VMEM((NP // 128, 128), jnp.float32),
            pltpu.VMEM((NCH, 128), jnp.int32),
            pltpu.VMEM((NCH, 128), jnp.int32),
            pltpu.VMEM((NP // 128, 128), jnp.float32),
            pltpu.VMEM((8, 128), jnp.float32),
            pltpu.VMEM((8, 128), jnp.float32),
            pltpu.VMEM_SHARED((16, NP // 128, 128), jnp.float32),
        ],
    )


def _sc_seg_scalar(vals, srcr, dstr):
    out = _sc_seg_scalar_call()(vals.reshape(NP // 128, 128), srcr, dstr)
    return out.reshape(2, NP)


def _tc1_body(x_ref, w_ref, d0_ref, d1_ref, a_ref, p_ref, dinv_ref):
    a = a_ref[...]
    deg = a * (d0_ref[...] + d1_ref[...]) + a
    dinv = jnp.where(deg > 0, 1.0 / jnp.sqrt(jnp.maximum(deg, 1e-12)), 0.0)
    h = jnp.dot(x_ref[...], w_ref[...], preferred_element_type=jnp.float32)
    p_ref[...] = dinv[:, None] * h
    dinv_ref[...] = dinv


_tc1 = pl.pallas_call(
    _tc1_body,
    out_shape=[jax.ShapeDtypeStruct((NP, H), jnp.float32),
               jax.ShapeDtypeStruct((NP,), jnp.float32)],
)


def _tc2_body(g0_ref, g1_ref, p_ref, dinv_ref, bc_ref, ws_ref, x1_ref, psc_ref):
    dinv = dinv_ref[...]
    pre = dinv[:, None] * (g0_ref[...] + g1_ref[...] + p_ref[...]) + bc_ref[...][None, :]
    x1 = jnp.maximum(pre, 0.0)
    x1_ref[...] = x1
    hsc = jnp.dot(x1, ws_ref[...], preferred_element_type=jnp.float32)
    psc_ref[...] = (dinv[:, None] * hsc)[:, 0:1]


_tc2 = pl.pallas_call(
    _tc2_body,
    out_shape=[jax.ShapeDtypeStruct((NP, H), jnp.float32),
               jax.ShapeDtypeStruct((NP, 1), jnp.float32)],
)


def _tc3_body(s0_ref, s1_ref, psc_ref, dinv_ref, bs_ref, a_ref, b_ref,
              score_ref, seff_ref, kg_ref):
    score = dinv_ref[...] * (s0_ref[...] + s1_ref[...] + psc_ref[...]) + bs_ref[0]
    score_ref[...] = score
    alive = a_ref[...] > 0
    seff_ref[...] = jnp.where(alive, score, -1e30)
    gids = lax.broadcasted_iota(jnp.int32, (G, NP), 0)
    onehot = jnp.where(gids == b_ref[...][None, :], 1.0, 0.0)
    cnt_alive = jnp.dot(onehot, a_ref[...], preferred_element_type=jnp.float32)
    kg_ref[...] = jnp.ceil(RATIO * cnt_alive)


_tc3 = pl.pallas_call(
    _tc3_body,
    out_shape=[jax.ShapeDtypeStruct((NP,), jnp.float32),
               jax.ShapeDtypeStruct((NP,), jnp.float32),
               jax.ShapeDtypeStruct((G,), jnp.float32)],
)


def _tc4a_body(seff_ref, b_ref, rank_ref):
    s = seff_ref[...]
    b = b_ref[...]
    ii = lax.broadcasted_iota(jnp.int32, (NP, JC), 0)
    jj0 = lax.broadcasted_iota(jnp.int32, (NP, JC), 1)

    def chunk(c, rank):
        jb = b_ref[pl.ds(c * JC, JC)]
        js = seff_ref[pl.ds(c * JC, JC)]
        same = b[:, None] == jb[None, :]
        sgt = js[None, :] > s[:, None]
        seq = js[None, :] == s[:, None]
        jlt = (jj0 + c * JC) < ii
        m = same & (sgt | (seq & jlt))
        return rank + jnp.sum(m.astype(jnp.float32), axis=1)

    rank_ref[...] = lax.fori_loop(0, NJC, chunk, jnp.zeros((NP,), jnp.float32))


_tc4a = pl.pallas_call(
    _tc4a_body,
    out_shape=jax.ShapeDtypeStruct((NP,), jnp.float32),
)


def _tc4b_body(rank_ref, kg_ref, a_ref, b_ref, score_ref, x1_ref,
               keep_ref, x2_ref):
    b = b_ref[...]
    gids = lax.broadcasted_iota(jnp.int32, (NP, G), 1)
    onehot = jnp.where(gids == b[:, None], 1.0, 0.0)
    kgb = jnp.dot(onehot, kg_ref[...], preferred_element_type=jnp.float32)
    keep = jnp.where((a_ref[...] > 0) & (rank_ref[...] < kgb), 1.0, 0.0)
    keep_ref[...] = keep
    x2_ref[...] = (keep * jnp.tanh(score_ref[...]))[:, None] * x1_ref[...]


_tc4b = pl.pallas_call(
    _tc4b_body,
    out_shape=[jax.ShapeDtypeStruct((NP,), jnp.float32),
               jax.ShapeDtypeStruct((NP, H), jnp.float32)],
)


def _tc5_body(x2_ref, keep_ref, b_ref, hout_ref):
    keep = keep_ref[...]
    b = b_ref[...]
    x2 = x2_ref[...]
    rows_mx = []
    rows_add = []
    for g in range(G):
        gm = jnp.where(b == g, 1.0, 0.0)
        mf = gm * keep
        masked = jnp.where(mf[:, None] > 0, x2, -1e30)
        rows_mx.append(jnp.max(masked, axis=0)[None, :])
        rows_add.append(jnp.sum(x2 * gm[:, None], axis=0)[None, :])
    mx_all = jnp.concatenate(rows_mx, axis=0)
    hout_ref[:, 0:H] = jnp.where(mx_all < -1e29, 0.0, mx_all)
    hout_ref[:, H:2 * H] = jnp.concatenate(rows_add, axis=0)


_tc5 = pl.pallas_call(
    _tc5_body,
    out_shape=jax.ShapeDtypeStruct((G, 2 * H), jnp.float32),
)


def _tc6_body(h0_ref, h1_ref, h2_ref, wf_ref, bf_ref, w0_ref, b0_ref,
              w1_ref, b1_ref, w2_ref, b2_ref, w3_ref, b3_ref, y_ref,
              out_ref, loss_ref):
    z = h0_ref[...] + h1_ref[...] + h2_ref[...]
    z = jnp.dot(z, wf_ref[...], preferred_element_type=jnp.float32) + bf_ref[...][None, :]
    z = jnp.maximum(jnp.dot(z, w0_ref[...], preferred_element_type=jnp.float32) + b0_ref[...][None, :], 0.0)
    z = jnp.maximum(jnp.dot(z, w1_ref[...], preferred_element_type=jnp.float32) + b1_ref[...][None, :], 0.0)
    z = jnp.maximum(jnp.dot(z, w2_ref[...], preferred_element_type=jnp.float32) + b2_ref[...][None, :], 0.0)
    out = jnp.dot(z, w3_ref[...], preferred_element_type=jnp.float32) + b3_ref[...][None, :]
    out_ref[...] = out
    d = out[:, 0] - y_ref[...]
    loss_ref[...] = jnp.sqrt(jnp.mean(d * d))[None, None]


_tc6 = pl.pallas_call(
    _tc6_body,
    out_shape=[jax.ShapeDtypeStruct((G, 8), jnp.float32),
               jax.ShapeDtypeStruct((1, 1), jnp.float32)],
)


def kernel(x, edge_index, batch, y, W0, b0, W1, b1, W2, b2, Ws0, bs0, Ws1, bs1,
           Ws2, bs2, Wf, bf, Wm0, bm0, Wm1, bm1, Wm2, bm2, Wm3, bm3):
    src = edge_index[0]
    dst = edge_index[1]
    pad_e = jnp.full((EP - E,), N, jnp.int32)
    srcr = jnp.concatenate([src, pad_e]).reshape(32, NCH, 128)
    dstr = jnp.concatenate([dst, pad_e]).reshape(32, NCH, 128)
    xp = jnp.pad(x, ((0, NP - N), (0, 0)))
    bp = jnp.pad(batch, (0, NP - N), constant_values=G - 1)
    a = (jnp.arange(NP) < N).astype(jnp.float32)
    yv = y.reshape(G)

    convs = [(W0, b0, jnp.pad(Ws0, ((0, 0), (0, H - 1))), bs0),
             (W1, b1, jnp.pad(Ws1, ((0, 0), (0, H - 1))), bs1),
             (W2, b2, jnp.pad(Ws2, ((0, 0), (0, H - 1))), bs2)]
    hs = []
    xc = xp
    for Wc, bc, WsP, bs in convs:
        degp = _sc_seg_scalar(a, srcr, dstr)
        p, dinv = _tc1(xc, Wc, degp[0], degp[1], a)
        aggp = _sc_agg_rows(p, srcr, dstr)
        x1, psc2 = _tc2(aggp[0], aggp[1], p, dinv, bc, WsP)
        psc = psc2.reshape(NP)
        aggsp = _sc_seg_scalar(psc, srcr, dstr)
        score, seff, kg = _tc3(aggsp[0], aggsp[1], psc, dinv, bs, a, bp)
        rank = _tc4a(seff, bp)
        keep, x2 = _tc4b(rank, kg, a, bp, score, x1)
        hs.append(_tc5(x2, keep, bp))
        xc, a = x2, keep

    Wm3p = jnp.pad(Wm3, ((0, 0), (0, 7)))
    bm3p = jnp.pad(bm3, (0, 7))
    out8, loss = _tc6(hs[0], hs[1], hs[2], Wf, bf, Wm0, bm0, Wm1, bm1,
                      Wm2, bm2, Wm3p, bm3p, yv)
    return (out8[:, :1], loss.reshape(()))


# speedup vs baseline: 21.6631x; 1.3970x over previous
import functools

import jax
import jax.numpy as jnp
from jax import lax
from jax.experimental import pallas as pl
from jax.experimental.pallas import tpu as pltpu
from jax.experimental.pallas import tpu_sc as plsc

N = 10000
NP = 10240
E = 320000
EPW = 10112
EP = EPW * 32
NCH = EPW // 128
G = 64
H = 128
RATIO = 0.5
JC = 256
NJC = NP // JC
IBS = 256
NIB = NP // IBS


def _sc_agg_rows_body(p_hbm, srcr, dstr, out_hbm, src_v, dst_v, rows_v, acc_sh,
                      sem0, sem1):
    cid = lax.axis_index("c")
    sid = lax.axis_index("s")
    wid = cid * 16 + sid
    sems = (sem0, sem1)
    z16 = jnp.zeros((16,), jnp.float32)

    def zrow(i, _):
        for j in range(8):
            rows_v[0, i, j * 16:(j + 1) * 16] = z16
        return 0

    lax.fori_loop(0, 128, zrow, 0)
    for t in range(5):
        pltpu.sync_copy(rows_v.at[0], acc_sh.at[pl.ds(sid * 640 + t * 128, 128)])
    plsc.subcore_barrier()

    pltpu.sync_copy(srcr.at[wid], src_v)
    pltpu.sync_copy(dstr.at[wid], dst_v)

    def body(c, _):
        pltpu.async_copy(p_hbm.at[src_v.at[c]], rows_v.at[0], sem0).wait()
        pltpu.sync_copy(rows_v.at[0], acc_sh.at[dst_v.at[c]], add=True)
        return 0

    lax.fori_loop(0, NCH, body, 0)
    plsc.subcore_barrier()
    for t in range(5):
        pltpu.sync_copy(acc_sh.at[pl.ds(sid * 640 + t * 128, 128)],
                        out_hbm.at[cid, pl.ds(sid * 640 + t * 128, 128)])


@functools.lru_cache(maxsize=None)
def _sc_agg_rows_call():
    mesh = plsc.VectorSubcoreMesh(core_axis_name="c", subcore_axis_name="s")
    return pl.kernel(
        _sc_agg_rows_body,
        out_type=jax.ShapeDtypeStruct((2, NP, H), jnp.float32),
        mesh=mesh,
        compiler_params=pltpu.CompilerParams(needs_layout_passes=False),
        scratch_types=[
            pltpu.VMEM((NCH, 128), jnp.int32),
            pltpu.VMEM((NCH, 128), jnp.int32),
            pltpu.VMEM((1, 128, H), jnp.float32),
            pltpu.VMEM_SHARED((NP, H), jnp.float32),
            pltpu.SemaphoreType.DMA,
            pltpu.SemaphoreType.DMA,
        ],
    )


def _sc_agg_rows(p, srcr, dstr):
    return _sc_agg_rows_call()(p, srcr, dstr)


def _sc_seg_scalar_body(vals_hbm, srcr, dstr, out_hbm, vals_v, src_v, dst_v,
                        part_v, tmp_v, red_v, stage_sh):
    cid = lax.axis_index("c")
    sid = lax.axis_index("s")
    wid = cid * 16 + sid
    z16 = jnp.zeros((16,), jnp.float32)

    def zpart(i, _):
        for j in range(8):
            part_v[i, j * 16:(j + 1) * 16] = z16
        return 0

    lax.fori_loop(0, NP // 128, zpart, 0)
    pltpu.sync_copy(vals_hbm, vals_v)
    pltpu.sync_copy(srcr.at[wid], src_v)
    pltpu.sync_copy(dstr.at[wid], dst_v)

    def body(c, _):
        for j in range(8):
            s16 = src_v[c, j * 16:(j + 1) * 16]
            d16 = dst_v[c, j * 16:(j + 1) * 16]
            v = plsc.load_gather(
                vals_v, [jnp.right_shift(s16, 7), jnp.bitwise_and(s16, 127)])
            plsc.addupdate_scatter(
                part_v, [jnp.right_shift(d16, 7), jnp.bitwise_and(d16, 127)], v)
        return 0

    lax.fori_loop(0, NCH, body, 0)
    pltpu.sync_copy(part_v, stage_sh.at[sid])
    plsc.subcore_barrier()

    @pl.when(sid < 10)
    def _():
        for i in range(8):
            for j in range(8):
                red_v[i, j * 16:(j + 1) * 16] = z16

        def comb(t, _):
            pltpu.sync_copy(stage_sh.at[t, pl.ds(sid * 8, 8)], tmp_v)
            for i in range(8):
                for j in range(8):
                    sl = (i, slice(j * 16, (j + 1) * 16))
                    red_v[sl] = red_v[sl] + tmp_v[sl]
            return 0

        lax.fori_loop(0, 16, comb, 0)
        pltpu.sync_copy(red_v, out_hbm.at[cid, pl.ds(sid * 8, 8)])


@functools.lru_cache(maxsize=None)
def _sc_seg_scalar_call():
    mesh = plsc.VectorSubcoreMesh(core_axis_name="c", subcore_axis_name="s")
    return pl.kernel(
        _sc_seg_scalar_body,
        out_type=jax.ShapeDtypeStruct((2, NP // 128, 128), jnp.float32),
        mesh=mesh,
        compiler_params=pltpu.CompilerParams(needs_layout_passes=False),
        scratch_types=[
            pltpu.VMEM((NP // 128, 128), jnp.float32),
            pltpu.VMEM((NCH, 128), jnp.int32),
            pltpu.VMEM((NCH, 128), jnp.int32),
            pltpu.VMEM((NP // 128, 128), jnp.float32),
            pltpu.VMEM((8, 128), jnp.float32),
            pltpu.VMEM((8, 128), jnp.float32),
            pltpu.VMEM_SHARED((16, NP // 128, 128), jnp.float32),
        ],
    )


def _sc_seg_scalar(vals, srcr, dstr):
    out = _sc_seg_scalar_call()(vals.reshape(NP // 128, 128), srcr, dstr)
    return out.reshape(2, NP)


def _tc1_body(x_ref, w_ref, d0_ref, d1_ref, a_ref, p_ref, dinv_ref):
    a = a_ref[...]
    deg = a * (d0_ref[...] + d1_ref[...]) + a
    dinv = jnp.where(deg > 0, 1.0 / jnp.sqrt(jnp.maximum(deg, 1e-12)), 0.0)
    h = jnp.dot(x_ref[...], w_ref[...], preferred_element_type=jnp.float32)
    p_ref[...] = dinv[:, None] * h
    dinv_ref[...] = dinv


_tc1 = pl.pallas_call(
    _tc1_body,
    out_shape=[jax.ShapeDtypeStruct((NP, H), jnp.float32),
               jax.ShapeDtypeStruct((NP,), jnp.float32)],
)


def _tc2_body(g0_ref, g1_ref, p_ref, dinv_ref, bc_ref, ws_ref, x1_ref, psc_ref):
    dinv = dinv_ref[...]
    pre = dinv[:, None] * (g0_ref[...] + g1_ref[...] + p_ref[...]) + bc_ref[...][None, :]
    x1 = jnp.maximum(pre, 0.0)
    x1_ref[...] = x1
    hsc = jnp.dot(x1, ws_ref[...], preferred_element_type=jnp.float32)
    psc_ref[...] = (dinv[:, None] * hsc)[:, 0:1]


_tc2 = pl.pallas_call(
    _tc2_body,
    out_shape=[jax.ShapeDtypeStruct((NP, H), jnp.float32),
               jax.ShapeDtypeStruct((NP, 1), jnp.float32)],
)


def _tc3_body(s0_ref, s1_ref, psc_ref, dinv_ref, bs_ref, a_ref, b_ref, b2_ref,
              score_ref, seff_ref, kg_ref, jlo_ref, jhi_ref):
    score = dinv_ref[...] * (s0_ref[...] + s1_ref[...] + psc_ref[...]) + bs_ref[0]
    score_ref[...] = score
    alive = a_ref[...] > 0
    seff_ref[...] = jnp.where(alive, score, -1e30)
    gids = lax.broadcasted_iota(jnp.int32, (G, NP), 0)
    onehot = jnp.where(gids == b_ref[...][None, :], 1.0, 0.0)
    cnt_alive = jnp.dot(onehot, a_ref[...], preferred_element_type=jnp.float32)
    kg_ref[...] = jnp.ceil(RATIO * cnt_alive)
    cnt_all = jnp.sum(onehot, axis=1)
    rr = lax.broadcasted_iota(jnp.int32, (G, G), 0)
    cc = lax.broadcasted_iota(jnp.int32, (G, G), 1)
    tril = jnp.where(cc < rr, 1.0, 0.0)
    starts = jnp.dot(tril, cnt_all, preferred_element_type=jnp.float32)
    ends = starts + cnt_all
    gids2 = lax.broadcasted_iota(jnp.int32, (NIB, G), 1)
    oh_f = jnp.where(gids2 == b2_ref[:, 0][:, None], 1.0, 0.0)
    oh_l = jnp.where(gids2 == b2_ref[:, IBS - 1][:, None], 1.0, 0.0)
    jlo_ref[...] = jnp.dot(oh_f, starts, preferred_element_type=jnp.float32).astype(jnp.int32)
    jhi_ref[...] = jnp.dot(oh_l, ends, preferred_element_type=jnp.float32).astype(jnp.int32)


_tc3 = pl.pallas_call(
    _tc3_body,
    out_shape=[jax.ShapeDtypeStruct((NP,), jnp.float32),
               jax.ShapeDtypeStruct((NP,), jnp.float32),
               jax.ShapeDtypeStruct((G,), jnp.float32),
               jax.ShapeDtypeStruct((NIB,), jnp.int32),
               jax.ShapeDtypeStruct((NIB,), jnp.int32)],
)


def _tc4a_body(seff_ref, b_ref, jlo_ref, jhi_ref, rank_ref):
    ib = pl.program_id(0)
    s = seff_ref[pl.ds(ib * IBS, IBS)]
    b = b_ref[pl.ds(ib * IBS, IBS)]
    lo = jlo_ref[ib]
    hi = jhi_ref[ib]
    c0 = lo // JC
    c1 = (hi + JC - 1) // JC
    ii = ib * IBS + lax.broadcasted_iota(jnp.int32, (IBS, JC), 0)
    jj0 = lax.broadcasted_iota(jnp.int32, (IBS, JC), 1)

    def chunk(c, rank):
        jb = b_ref[pl.ds(c * JC, JC)]
        js = seff_ref[pl.ds(c * JC, JC)]
        same = b[:, None] == jb[None, :]
        sgt = js[None, :] > s[:, None]
        seq = js[None, :] == s[:, None]
        jlt = (jj0 + c * JC) < ii
        m = same & (sgt | (seq & jlt))
        return rank + jnp.sum(m.astype(jnp.float32), axis=1)

    rank_ref[...] = lax.fori_loop(c0, c1, chunk, jnp.zeros((IBS,), jnp.float32))


_tc4a = pl.pallas_call(
    _tc4a_body,
    grid=(NIB,),
    in_specs=[
        pl.BlockSpec((NP,), lambda i: (0,)),
        pl.BlockSpec((NP,), lambda i: (0,)),
        pl.BlockSpec(memory_space=pltpu.SMEM),
        pl.BlockSpec(memory_space=pltpu.SMEM),
    ],
    out_specs=pl.BlockSpec((IBS,), lambda i: (i,)),
    out_shape=jax.ShapeDtypeStruct((NP,), jnp.float32),
)


def _tc4b_body(rank_ref, kg_ref, a_ref, b_ref, score_ref, x1_ref,
               keep_ref, x2_ref):
    b = b_ref[...]
    gids = lax.broadcasted_iota(jnp.int32, (NP, G), 1)
    onehot = jnp.where(gids == b[:, None], 1.0, 0.0)
    kgb = jnp.dot(onehot, kg_ref[...], preferred_element_type=jnp.float32)
    keep = jnp.where((a_ref[...] > 0) & (rank_ref[...] < kgb), 1.0, 0.0)
    keep_ref[...] = keep
    x2_ref[...] = (keep * jnp.tanh(score_ref[...]))[:, None] * x1_ref[...]


_tc4b = pl.pallas_call(
    _tc4b_body,
    out_shape=[jax.ShapeDtypeStruct((NP,), jnp.float32),
               jax.ShapeDtypeStruct((NP, H), jnp.float32)],
)


def _tc5_body(x2_ref, keep_ref, b_ref, hout_ref):
    keep = keep_ref[...]
    b = b_ref[...]
    x2 = x2_ref[...]
    rows_mx = []
    rows_add = []
    for g in range(G):
        gm = jnp.where(b == g, 1.0, 0.0)
        mf = gm * keep
        masked = jnp.where(mf[:, None] > 0, x2, -1e30)
        rows_mx.append(jnp.max(masked, axis=0)[None, :])
        rows_add.append(jnp.sum(x2 * gm[:, None], axis=0)[None, :])
    mx_all = jnp.concatenate(rows_mx, axis=0)
    hout_ref[:, 0:H] = jnp.where(mx_all < -1e29, 0.0, mx_all)
    hout_ref[:, H:2 * H] = jnp.concatenate(rows_add, axis=0)


_tc5 = pl.pallas_call(
    _tc5_body,
    out_shape=jax.ShapeDtypeStruct((G, 2 * H), jnp.float32),
)


def _tc6_body(h0_ref, h1_ref, h2_ref, wf_ref, bf_ref, w0_ref, b0_ref,
              w1_ref, b1_ref, w2_ref, b2_ref, w3_ref, b3_ref, y_ref,
              out_ref, loss_ref):
    z = h0_ref[...] + h1_ref[...] + h2_ref[...]
    z = jnp.dot(z, wf_ref[...], preferred_element_type=jnp.float32) + bf_ref[...][None, :]
    z = jnp.maximum(jnp.dot(z, w0_ref[...], preferred_element_type=jnp.float32) + b0_ref[...][None, :], 0.0)
    z = jnp.maximum(jnp.dot(z, w1_ref[...], preferred_element_type=jnp.float32) + b1_ref[...][None, :], 0.0)
    z = jnp.maximum(jnp.dot(z, w2_ref[...], preferred_element_type=jnp.float32) + b2_ref[...][None, :], 0.0)
    out = jnp.dot(z, w3_ref[...], preferred_element_type=jnp.float32) + b3_ref[...][None, :]
    out_ref[...] = out
    d = out[:, 0] - y_ref[...]
    loss_ref[...] = jnp.sqrt(jnp.mean(d * d))[None, None]


_tc6 = pl.pallas_call(
    _tc6_body,
    out_shape=[jax.ShapeDtypeStruct((G, 8), jnp.float32),
               jax.ShapeDtypeStruct((1, 1), jnp.float32)],
)


def kernel(x, edge_index, batch, y, W0, b0, W1, b1, W2, b2, Ws0, bs0, Ws1, bs1,
           Ws2, bs2, Wf, bf, Wm0, bm0, Wm1, bm1, Wm2, bm2, Wm3, bm3):
    src = edge_index[0]
    dst = edge_index[1]
    pad_e = jnp.full((EP - E,), N, jnp.int32)
    srcr = jnp.concatenate([src, pad_e]).reshape(32, NCH, 128)
    dstr = jnp.concatenate([dst, pad_e]).reshape(32, NCH, 128)
    xp = jnp.pad(x, ((0, NP - N), (0, 0)))
    bp = jnp.pad(batch, (0, NP - N), constant_values=G - 1)
    a = (jnp.arange(NP) < N).astype(jnp.float32)
    yv = y.reshape(G)

    convs = [(W0, b0, jnp.pad(Ws0, ((0, 0), (0, H - 1))), bs0),
             (W1, b1, jnp.pad(Ws1, ((0, 0), (0, H - 1))), bs1),
             (W2, b2, jnp.pad(Ws2, ((0, 0), (0, H - 1))), bs2)]
    hs = []
    xc = xp
    for Wc, bc, WsP, bs in convs:
        degp = _sc_seg_scalar(a, srcr, dstr)
        p, dinv = _tc1(xc, Wc, degp[0], degp[1], a)
        aggp = _sc_agg_rows(p, srcr, dstr)
        x1, psc2 = _tc2(aggp[0], aggp[1], p, dinv, bc, WsP)
        psc = psc2.reshape(NP)
        aggsp = _sc_seg_scalar(psc, srcr, dstr)
        score, seff, kg, jlo, jhi = _tc3(aggsp[0], aggsp[1], psc, dinv, bs, a,
                                         bp, bp.reshape(NIB, IBS))
        rank = _tc4a(seff, bp, jlo, jhi)
        keep, x2 = _tc4b(rank, kg, a, bp, score, x1)
        hs.append(_tc5(x2, keep, bp))
        xc, a = x2, keep

    Wm3p = jnp.pad(Wm3, ((0, 0), (0, 7)))
    bm3p = jnp.pad(bm3, (0, 7))
    out8, loss = _tc6(hs[0], hs[1], hs[2], Wf, bf, Wm0, bm0, Wm1, bm1,
                      Wm2, bm2, Wm3p, bm3p, yv)
    return (out8[:, :1], loss.reshape(()))


# speedup vs baseline: 21.6648x; 1.0001x over previous
import functools

import jax
import jax.numpy as jnp
from jax import lax
from jax.experimental import pallas as pl
from jax.experimental.pallas import tpu as pltpu
from jax.experimental.pallas import tpu_sc as plsc

N = 10000
NP = 10240
E = 320000
EPW = 10112
EP = EPW * 32
NCH = EPW // 128
NPA = 10112
G = 64
H = 128
RATIO = 0.5
JC = 256
NJC = NP // JC
IBS = 256
NIB = NP // IBS


def _sc_agg_rows_body(p_hbm, srcr, dstr, out_hbm, src_v, dst_v, rows_v, acc_sh,
                      sem0, sem1):
    cid = lax.axis_index("c")
    sid = lax.axis_index("s")
    wid = cid * 16 + sid
    sems = (sem0, sem1)
    z16 = jnp.zeros((16,), jnp.float32)

    def zrow(i, _):
        for j in range(8):
            rows_v[0, i, j * 16:(j + 1) * 16] = z16
        return 0

    lax.fori_loop(0, 128, zrow, 0)
    for t in range(4):
        pltpu.sync_copy(rows_v.at[0], acc_sh.at[pl.ds(sid * 632 + t * 128, 128)])
    pltpu.sync_copy(rows_v.at[0, 0:120], acc_sh.at[pl.ds(sid * 632 + 512, 120)])
    plsc.subcore_barrier()

    pltpu.sync_copy(srcr.at[wid], src_v)
    pltpu.sync_copy(dstr.at[wid], dst_v)

    def body(c, _):
        pltpu.async_copy(p_hbm.at[src_v.at[c]], rows_v.at[0], sem0).wait()
        pltpu.sync_copy(rows_v.at[0], acc_sh.at[dst_v.at[c]], add=True)
        return 0

    lax.fori_loop(0, NCH, body, 0)
    plsc.subcore_barrier()
    for t in range(4):
        pltpu.sync_copy(acc_sh.at[pl.ds(sid * 632 + t * 128, 128)],
                        out_hbm.at[cid, pl.ds(sid * 632 + t * 128, 128)])
    pltpu.sync_copy(acc_sh.at[pl.ds(sid * 632 + 512, 120)],
                    out_hbm.at[cid, pl.ds(sid * 632 + 512, 120)])


@functools.lru_cache(maxsize=None)
def _sc_agg_rows_call():
    mesh = plsc.VectorSubcoreMesh(core_axis_name="c", subcore_axis_name="s")
    return pl.kernel(
        _sc_agg_rows_body,
        out_type=jax.ShapeDtypeStruct((2, NPA, H), jnp.float32),
        mesh=mesh,
        compiler_params=pltpu.CompilerParams(needs_layout_passes=False),
        scratch_types=[
            pltpu.VMEM((NCH, 128), jnp.int32),
            pltpu.VMEM((NCH, 128), jnp.int32),
            pltpu.VMEM((1, 128, H), jnp.float32),
            pltpu.VMEM_SHARED((NPA, H), jnp.float32),
            pltpu.SemaphoreType.DMA,
            pltpu.SemaphoreType.DMA,
        ],
    )


def _sc_agg_rows(p, srcr, dstr):
    return _sc_agg_rows_call()(p, srcr, dstr)


def _sc_seg_scalar_body(vals_hbm, srcr, dstr, out_hbm, vals_v, src_v, dst_v,
                        part_v, tmp_v, red_v, stage_sh):
    cid = lax.axis_index("c")
    sid = lax.axis_index("s")
    wid = cid * 16 + sid
    z16 = jnp.zeros((16,), jnp.float32)

    def zpart(i, _):
        for j in range(8):
            part_v[i, j * 16:(j + 1) * 16] = z16
        return 0

    lax.fori_loop(0, NP // 128, zpart, 0)
    pltpu.sync_copy(vals_hbm, vals_v)
    pltpu.sync_copy(srcr.at[wid], src_v)
    pltpu.sync_copy(dstr.at[wid], dst_v)

    def body(c, _):
        for j in range(8):
            s16 = src_v[c, j * 16:(j + 1) * 16]
            d16 = dst_v[c, j * 16:(j + 1) * 16]
            v = plsc.load_gather(
                vals_v, [jnp.right_shift(s16, 7), jnp.bitwise_and(s16, 127)])
            plsc.addupdate_scatter(
                part_v, [jnp.right_shift(d16, 7), jnp.bitwise_and(d16, 127)], v)
        return 0

    lax.fori_loop(0, NCH, body, 0)
    pltpu.sync_copy(part_v, stage_sh.at[sid])
    plsc.subcore_barrier()

    @pl.when(sid < 10)
    def _():
        for i in range(8):
            for j in range(8):
                red_v[i, j * 16:(j + 1) * 16] = z16

        def comb(t, _):
            pltpu.sync_copy(stage_sh.at[t, pl.ds(sid * 8, 8)], tmp_v)
            for i in range(8):
                for j in range(8):
                    sl = (i, slice(j * 16, (j + 1) * 16))
                    red_v[sl] = red_v[sl] + tmp_v[sl]
            return 0

        lax.fori_loop(0, 16, comb, 0)
        pltpu.sync_copy(red_v, out_hbm.at[cid, pl.ds(sid * 8, 8)])


@functools.lru_cache(maxsize=None)
def _sc_seg_scalar_call():
    mesh = plsc.VectorSubcoreMesh(core_axis_name="c", subcore_axis_name="s")
    return pl.kernel(
        _sc_seg_scalar_body,
        out_type=jax.ShapeDtypeStruct((2, NP // 128, 128), jnp.float32),
        mesh=mesh,
        compiler_params=pltpu.CompilerParams(needs_layout_passes=False),
        scratch_types=[
            pltpu.VMEM((NP // 128, 128), jnp.float32),
            pltpu.VMEM((NCH, 128), jnp.int32),
            pltpu.VMEM((NCH, 128), jnp.int32),
            pltpu.VMEM((NP // 128, 128), jnp.float32),
            pltpu.VMEM((8, 128), jnp.float32),
            pltpu.VMEM((8, 128), jnp.float32),
            pltpu.VMEM_SHARED((16, NP // 128, 128), jnp.float32),
        ],
    )


def _sc_seg_scalar(vals, srcr, dstr):
    out = _sc_seg_scalar_call()(vals.reshape(NP // 128, 128), srcr, dstr)
    return out.reshape(2, NP)


def _tc1_body(x_ref, w_ref, d0_ref, d1_ref, a_ref, p_ref, dinv_ref):
    a = a_ref[...]
    deg = a * (d0_ref[...] + d1_ref[...]) + a
    dinv = jnp.where(deg > 0, 1.0 / jnp.sqrt(jnp.maximum(deg, 1e-12)), 0.0)
    h = jnp.dot(x_ref[...], w_ref[...], preferred_element_type=jnp.float32)
    p_ref[...] = dinv[:, None] * h
    dinv_ref[...] = dinv


_tc1 = pl.pallas_call(
    _tc1_body,
    out_shape=[jax.ShapeDtypeStruct((NP, H), jnp.float32),
               jax.ShapeDtypeStruct((NP,), jnp.float32)],
)


def _tc2_body(g0_ref, g1_ref, p_ref, dinv_ref, bc_ref, ws_ref, x1_ref, psc_ref):
    dinv = dinv_ref[...]
    gg = jnp.concatenate(
        [g0_ref[...] + g1_ref[...], jnp.zeros((NP - NPA, H), jnp.float32)], axis=0)
    pre = dinv[:, None] * (gg + p_ref[...]) + bc_ref[...][None, :]
    x1 = jnp.maximum(pre, 0.0)
    x1_ref[...] = x1
    hsc = jnp.dot(x1, ws_ref[...], preferred_element_type=jnp.float32)
    psc_ref[...] = (dinv[:, None] * hsc)[:, 0:1]


_tc2 = pl.pallas_call(
    _tc2_body,
    out_shape=[jax.ShapeDtypeStruct((NP, H), jnp.float32),
               jax.ShapeDtypeStruct((NP, 1), jnp.float32)],
)


def _tc3_body(s0_ref, s1_ref, psc_ref, dinv_ref, bs_ref, a_ref, b_ref, b2_ref,
              score_ref, seff_ref, kg_ref, jlo_ref, jhi_ref):
    score = dinv_ref[...] * (s0_ref[...] + s1_ref[...] + psc_ref[...]) + bs_ref[0]
    score_ref[...] = score
    alive = a_ref[...] > 0
    seff_ref[...] = jnp.where(alive, score, -1e30)
    gids = lax.broadcasted_iota(jnp.int32, (G, NP), 0)
    onehot = jnp.where(gids == b_ref[...][None, :], 1.0, 0.0)
    cnt_alive = jnp.dot(onehot, a_ref[...], preferred_element_type=jnp.float32)
    kg_ref[...] = jnp.ceil(RATIO * cnt_alive)
    cnt_all = jnp.sum(onehot, axis=1)
    rr = lax.broadcasted_iota(jnp.int32, (G, G), 0)
    cc = lax.broadcasted_iota(jnp.int32, (G, G), 1)
    tril = jnp.where(cc < rr, 1.0, 0.0)
    starts = jnp.dot(tril, cnt_all, preferred_element_type=jnp.float32)
    ends = starts + cnt_all
    gids2 = lax.broadcasted_iota(jnp.int32, (NIB, G), 1)
    oh_f = jnp.where(gids2 == b2_ref[:, 0][:, None], 1.0, 0.0)
    oh_l = jnp.where(gids2 == b2_ref[:, IBS - 1][:, None], 1.0, 0.0)
    jlo_ref[...] = jnp.dot(oh_f, starts, preferred_element_type=jnp.float32).astype(jnp.int32)
    jhi_ref[...] = jnp.dot(oh_l, ends, preferred_element_type=jnp.float32).astype(jnp.int32)


_tc3 = pl.pallas_call(
    _tc3_body,
    out_shape=[jax.ShapeDtypeStruct((NP,), jnp.float32),
               jax.ShapeDtypeStruct((NP,), jnp.float32),
               jax.ShapeDtypeStruct((G,), jnp.float32),
               jax.ShapeDtypeStruct((NIB,), jnp.int32),
               jax.ShapeDtypeStruct((NIB,), jnp.int32)],
)


def _tc4a_body(seff_ref, b_ref, jlo_ref, jhi_ref, rank_ref):
    ib = pl.program_id(0)
    s = seff_ref[pl.ds(ib * IBS, IBS)]
    b = b_ref[pl.ds(ib * IBS, IBS)]
    lo = jlo_ref[ib]
    hi = jhi_ref[ib]
    c0 = lo // JC
    c1 = (hi + JC - 1) // JC
    ii = ib * IBS + lax.broadcasted_iota(jnp.int32, (IBS, JC), 0)
    jj0 = lax.broadcasted_iota(jnp.int32, (IBS, JC), 1)

    def chunk(c, rank):
        jb = b_ref[pl.ds(c * JC, JC)]
        js = seff_ref[pl.ds(c * JC, JC)]
        same = b[:, None] == jb[None, :]
        sgt = js[None, :] > s[:, None]
        seq = js[None, :] == s[:, None]
        jlt = (jj0 + c * JC) < ii
        m = same & (sgt | (seq & jlt))
        return rank + jnp.sum(m.astype(jnp.float32), axis=1)

    rank_ref[...] = lax.fori_loop(c0, c1, chunk, jnp.zeros((IBS,), jnp.float32))


_tc4a = pl.pallas_call(
    _tc4a_body,
    grid=(NIB,),
    in_specs=[
        pl.BlockSpec((NP,), lambda i: (0,)),
        pl.BlockSpec((NP,), lambda i: (0,)),
        pl.BlockSpec(memory_space=pltpu.SMEM),
        pl.BlockSpec(memory_space=pltpu.SMEM),
    ],
    out_specs=pl.BlockSpec((IBS,), lambda i: (i,)),
    out_shape=jax.ShapeDtypeStruct((NP,), jnp.float32),
)


def _tc4b_body(rank_ref, kg_ref, a_ref, b_ref, score_ref, x1_ref,
               keep_ref, x2_ref):
    b = b_ref[...]
    gids = lax.broadcasted_iota(jnp.int32, (NP, G), 1)
    onehot = jnp.where(gids == b[:, None], 1.0, 0.0)
    kgb = jnp.dot(onehot, kg_ref[...], preferred_element_type=jnp.float32)
    keep = jnp.where((a_ref[...] > 0) & (rank_ref[...] < kgb), 1.0, 0.0)
    keep_ref[...] = keep
    x2_ref[...] = (keep * jnp.tanh(score_ref[...]))[:, None] * x1_ref[...]


_tc4b = pl.pallas_call(
    _tc4b_body,
    out_shape=[jax.ShapeDtypeStruct((NP,), jnp.float32),
               jax.ShapeDtypeStruct((NP, H), jnp.float32)],
)


def _tc5_body(x2_ref, keep_ref, b_ref, hout_ref):
    keep = keep_ref[...]
    b = b_ref[...]
    x2 = x2_ref[...]
    rows_mx = []
    rows_add = []
    for g in range(G):
        gm = jnp.where(b == g, 1.0, 0.0)
        mf = gm * keep
        masked = jnp.where(mf[:, None] > 0, x2, -1e30)
        rows_mx.append(jnp.max(masked, axis=0)[None, :])
        rows_add.append(jnp.sum(x2 * gm[:, None], axis=0)[None, :])
    mx_all = jnp.concatenate(rows_mx, axis=0)
    hout_ref[:, 0:H] = jnp.where(mx_all < -1e29, 0.0, mx_all)
    hout_ref[:, H:2 * H] = jnp.concatenate(rows_add, axis=0)


_tc5 = pl.pallas_call(
    _tc5_body,
    out_shape=jax.ShapeDtypeStruct((G, 2 * H), jnp.float32),
)


def _tc6_body(h0_ref, h1_ref, h2_ref, wf_ref, bf_ref, w0_ref, b0_ref,
              w1_ref, b1_ref, w2_ref, b2_ref, w3_ref, b3_ref, y_ref,
              out_ref, loss_ref):
    z = h0_ref[...] + h1_ref[...] + h2_ref[...]
    z = jnp.dot(z, wf_ref[...], preferred_element_type=jnp.float32) + bf_ref[...][None, :]
    z = jnp.maximum(jnp.dot(z, w0_ref[...], preferred_element_type=jnp.float32) + b0_ref[...][None, :], 0.0)
    z = jnp.maximum(jnp.dot(z, w1_ref[...], preferred_element_type=jnp.float32) + b1_ref[...][None, :], 0.0)
    z = jnp.maximum(jnp.dot(z, w2_ref[...], preferred_element_type=jnp.float32) + b2_ref[...][None, :], 0.0)
    out = jnp.dot(z, w3_ref[...], preferred_element_type=jnp.float32) + b3_ref[...][None, :]
    out_ref[...] = out
    d = out[:, 0] - y_ref[...]
    loss_ref[...] = jnp.sqrt(jnp.mean(d * d))[None, None]


_tc6 = pl.pallas_call(
    _tc6_body,
    out_shape=[jax.ShapeDtypeStruct((G, 8), jnp.float32),
               jax.ShapeDtypeStruct((1, 1), jnp.float32)],
)


def kernel(x, edge_index, batch, y, W0, b0, W1, b1, W2, b2, Ws0, bs0, Ws1, bs1,
           Ws2, bs2, Wf, bf, Wm0, bm0, Wm1, bm1, Wm2, bm2, Wm3, bm3):
    src = edge_index[0]
    dst = edge_index[1]
    pad_e = jnp.full((EP - E,), N, jnp.int32)
    srcr = jnp.concatenate([src, pad_e]).reshape(32, NCH, 128)
    dstr = jnp.concatenate([dst, pad_e]).reshape(32, NCH, 128)
    xp = jnp.pad(x, ((0, NP - N), (0, 0)))
    bp = jnp.pad(batch, (0, NP - N), constant_values=G - 1)
    a = (jnp.arange(NP) < N).astype(jnp.float32)
    yv = y.reshape(G)

    convs = [(W0, b0, jnp.pad(Ws0, ((0, 0), (0, H - 1))), bs0),
             (W1, b1, jnp.pad(Ws1, ((0, 0), (0, H - 1))), bs1),
             (W2, b2, jnp.pad(Ws2, ((0, 0), (0, H - 1))), bs2)]
    hs = []
    xc = xp
    for Wc, bc, WsP, bs in convs:
        degp = _sc_seg_scalar(a, srcr, dstr)
        p, dinv = _tc1(xc, Wc, degp[0], degp[1], a)
        aggp = _sc_agg_rows(p, srcr, dstr)
        x1, psc2 = _tc2(aggp[0], aggp[1], p, dinv, bc, WsP)
        psc = psc2.reshape(NP)
        aggsp = _sc_seg_scalar(psc, srcr, dstr)
        score, seff, kg, jlo, jhi = _tc3(aggsp[0], aggsp[1], psc, dinv, bs, a,
                                         bp, bp.reshape(NIB, IBS))
        rank = _tc4a(seff, bp, jlo, jhi)
        keep, x2 = _tc4b(rank, kg, a, bp, score, x1)
        hs.append(_tc5(x2, keep, bp))
        xc, a = x2, keep

    Wm3p = jnp.pad(Wm3, ((0, 0), (0, 7)))
    bm3p = jnp.pad(bm3, (0, 7))
    out8, loss = _tc6(hs[0], hs[1], hs[2], Wf, bf, Wm0, bm0, Wm1, bm1,
                      Wm2, bm2, Wm3p, bm3p, yv)
    return (out8[:, :1], loss.reshape(()))
